# baseline (speedup 1.0000x reference)
import functools

import jax
import jax.numpy as jnp
from jax import lax
from jax.experimental import pallas as pl
from jax.experimental.pallas import tpu as pltpu
from jax.experimental.pallas import tpu_sc as plsc

_NC = 50
_PW = 5.0
_QP = 304


def _dense_body(x_ref, pp_ref, lab_ref, tp_ref, big_ref, bg_ref,
                xl_scr, ppt_scr):
    B, Q, C = x_ref.shape
    T = lab_ref.shape[1]
    D = pp_ref.shape[2]

    x = x_ref[...]
    m = jnp.max(x, axis=2, keepdims=True)
    e = jnp.exp(x - m)

    labs = lab_ref[...]
    ci = lax.broadcasted_iota(jnp.int32, (B, T + 2, C), 2)
    ri = lax.broadcasted_iota(jnp.int32, (B, T + 2, C), 1)
    oh_lab = (ci == jnp.pad(labs, ((0, 0), (0, 2)))[:, :, None]).astype(jnp.float32)
    oh_bg = (ci == _NC).astype(jnp.float32)
    sel = jnp.where(ri < T, oh_lab, jnp.where(ri == T, 1.0, oh_bg))
    for b in range(B):
        xl_scr[b] = lax.dot_general(sel[b], e[b], (((1,), (1,)), ((), ())),
                                    preferred_element_type=jnp.float32)
    exl = xl_scr[:, :T, :]
    s_row = xl_scr[:, T:T + 1, :]
    ebg = xl_scr[:, T + 1:T + 2, :]

    cls_cost = -(exl / s_row)
    logls = jnp.log(s_row)
    lp = jnp.log(exl) - logls
    bg_lp = jnp.log(ebg) - logls
    ce_bg = -jnp.sum(bg_lp)

    di = lax.broadcasted_iota(jnp.int32, (D, D), 0)
    dj = lax.broadcasted_iota(jnp.int32, (D, D), 1)
    ident = (di == dj).astype(jnp.float32)
    pp = pp_ref[...]
    for b in range(B):
        ppt_scr[b] = lax.dot_general(ident, pp[b], (((1,), (1,)), ((), ())),
                                     preferred_element_type=jnp.float32)

    tp = tp_ref[...]
    poly = jnp.zeros((B, T, Q), dtype=jnp.float32)
    for d in range(D):
        a_d = ppt_scr[:, d:d + 1, :]
        b_d = tp[:, :, d:d + 1]
        poly = poly + jnp.abs(a_d - b_d)

    big_ref[0, :, :, :Q] = cls_cost + _PW * poly
    big_ref[0, :, :, Q:] = jnp.full((B, T, _QP - Q), jnp.inf, jnp.float32)
    big_ref[1, :, :, :Q] = lp
    big_ref[1, :, :, Q:] = jnp.zeros((B, T, _QP - Q), jnp.float32)
    bg_ref[:, :, :Q] = bg_lp
    bg_ref[:, :, Q:] = jnp.zeros((B, 1, _QP - Q), jnp.float32)
    bg_ref[0:1, 0:1, _QP - 1:_QP] = jnp.reshape(ce_bg, (1, 1, 1))


def _sc_match(big, bg, ce_scale, poly_scale):
    _, B, T, QP = big.shape
    nj = QP // 16
    nw = T * QP
    mesh = plsc.VectorSubcoreMesh(core_axis_name="c", subcore_axis_name="s")
    big = big.reshape(2 * B * T * QP)
    bg = bg.reshape(B * QP)

    @functools.partial(
        pl.kernel, mesh=mesh,
        compiler_params=pltpu.CompilerParams(needs_layout_passes=False),
        out_type=[
            jax.ShapeDtypeStruct((B * 16,), jnp.float32),
            jax.ShapeDtypeStruct((16,), jnp.float32),
        ],
        scratch_types=[
            pltpu.VMEM((T * QP,), jnp.float32),
            pltpu.VMEM((T * QP,), jnp.float32),
            pltpu.VMEM((QP,), jnp.float32),
            pltpu.VMEM((QP,), jnp.float32),
            pltpu.VMEM((16,), jnp.float32),
        ],
    )
    def k(big_hbm, bg_hbm, part_hbm, fin_hbm, c_v, lp_v, bg_v, used_v, out_v):
        cid = lax.axis_index("c")
        sid = lax.axis_index("s")

        @pl.when(cid == 0)
        def _():
            w = sid
            pltpu.sync_copy(big_hbm.at[pl.ds(w * nw, nw)], c_v)
            pltpu.sync_copy(big_hbm.at[pl.ds(B * nw + w * nw, nw)], lp_v)
            pltpu.sync_copy(bg_hbm.at[pl.ds(w * QP, QP)], bg_v)
            il = lax.iota(jnp.int32, 16)
            for j in range(nj):
                used_v[pl.ds(16 * j, 16)] = jnp.zeros((16,), jnp.float32)

            def gstep(t, carry):
                pacc, cacc = carry
                base = QP * t
                bestv = jnp.full((16,), jnp.inf, jnp.float32)
                besti = jnp.full((16,), jnp.int32(2 ** 30))
                for j in range(nj):
                    cv = c_v[pl.ds(base + 16 * j, 16)]
                    uv = used_v[pl.ds(16 * j, 16)]
                    mv = cv + uv
                    upd = mv < bestv
                    bestv = jnp.where(upd, mv, bestv)
                    besti = jnp.where(upd, 16 * j + il, besti)
                mn = jnp.min(bestv)
                cand = jnp.where(bestv == mn, besti, jnp.int32(2 ** 30))
                i = jnp.min(cand)
                jstar = i // 16
                sel = il == (i - 16 * jstar)
                ustar = used_v[pl.ds(16 * jstar, 16)]
                used_v[pl.ds(16 * jstar, 16)] = jnp.where(
                    sel, jnp.float32(jnp.inf), ustar)
                lpc = lp_v[pl.ds(base + 16 * jstar, 16)]
                bgc = bg_v[pl.ds(16 * jstar, 16)]
                pacc = pacc + jnp.where(
                    sel, (mn + jnp.exp(lpc)) * jnp.float32(1.0 / _PW), 0.0)
                cacc = cacc + jnp.where(sel, bgc - lpc, 0.0)
                return pacc, cacc

            z16 = jnp.zeros((16,), jnp.float32)
            pacc, cacc = lax.fori_loop(0, T, gstep, (z16, z16))
            last = bg_v[pl.ds(QP - 16, 16)]
            cacc = cacc + jnp.where(il == 15, last, 0.0)
            csum = jnp.sum(cacc)
            psum = jnp.sum(pacc)
            out_v[...] = jnp.where(il == 0, csum,
                                   jnp.where(il == 1, psum, 0.0))
            pltpu.sync_copy(out_v, part_hbm.at[pl.ds(w * 16, 16)])
            plsc.subcore_barrier()

            @pl.when(sid == 0)
            def _():
                acc = jnp.zeros((16,), jnp.float32)
                for g in range(B):
                    pltpu.sync_copy(part_hbm.at[pl.ds(g * 16, 16)], out_v)
                    acc = acc + out_v[...]
                scale = jnp.where(il == 0, jnp.float32(ce_scale),
                                  jnp.where(il == 1, jnp.float32(poly_scale),
                                            0.0))
                out_v[...] = acc * scale
                pltpu.sync_copy(out_v, fin_hbm)

    return k(big, bg)


def kernel(pred_logits, pred_polylines, tgt_labels, tgt_polylines):
    B, Q, C = pred_logits.shape
    T = tgt_labels.shape[1]
    big, bgm = pl.pallas_call(
        _dense_body,
        out_shape=[
            jax.ShapeDtypeStruct((2, B, T, _QP), jnp.float32),
            jax.ShapeDtypeStruct((B, 1, _QP), jnp.float32),
        ],
        scratch_shapes=[
            pltpu.VMEM((B, T + 2, Q), jnp.float32),
            pltpu.VMEM((B, pred_polylines.shape[2], Q), jnp.float32),
        ],
    )(pred_logits, pred_polylines, tgt_labels.astype(jnp.int32), tgt_polylines)
    _, fin = _sc_match(big, bgm, 1.0 / (B * Q), 1.0 / (B * T))
    return fin[:2]


# speedup vs baseline: 1.2157x; 1.2157x over previous
import functools

import jax
import jax.numpy as jnp
from jax import lax
from jax.experimental import pallas as pl
from jax.experimental.pallas import tpu as pltpu
from jax.experimental.pallas import tpu_sc as plsc

_NC = 50
_PW = 5.0
_QP = 304


def _dense_body(x_ref, pp_ref, lab_ref, tp_ref, big_ref, bg_ref,
                xl_scr, ppt_scr):
    B, Q, C = x_ref.shape
    T = lab_ref.shape[1]
    D = pp_ref.shape[2]

    x = x_ref[...]
    m = jnp.max(x, axis=2, keepdims=True)
    e = jnp.exp(x - m)

    labs = lab_ref[...]
    ci = lax.broadcasted_iota(jnp.int32, (B, T + 2, C), 2)
    ri = lax.broadcasted_iota(jnp.int32, (B, T + 2, C), 1)
    oh_lab = (ci == jnp.pad(labs, ((0, 0), (0, 2)))[:, :, None]).astype(jnp.float32)
    oh_bg = (ci == _NC).astype(jnp.float32)
    sel = jnp.where(ri < T, oh_lab, jnp.where(ri == T, 1.0, oh_bg))
    for b in range(B):
        xl_scr[b] = lax.dot_general(sel[b], e[b], (((1,), (1,)), ((), ())),
                                    preferred_element_type=jnp.float32)
    exl = xl_scr[:, :T, :]
    s_row = xl_scr[:, T:T + 1, :]
    ebg = xl_scr[:, T + 1:T + 2, :]

    cls_cost = -(exl / s_row)
    logls = jnp.log(s_row)
    lp = jnp.log(exl) - logls
    bg_lp = jnp.log(ebg) - logls
    ce_bg = -jnp.sum(bg_lp)

    di = lax.broadcasted_iota(jnp.int32, (D, D), 0)
    dj = lax.broadcasted_iota(jnp.int32, (D, D), 1)
    ident = (di == dj).astype(jnp.float32)
    pp = pp_ref[...]
    for b in range(B):
        ppt_scr[b] = lax.dot_general(ident, pp[b], (((1,), (1,)), ((), ())),
                                     preferred_element_type=jnp.float32)

    tp = tp_ref[...]
    poly = jnp.zeros((B, T, Q), dtype=jnp.float32)
    for d in range(D):
        a_d = ppt_scr[:, d:d + 1, :]
        b_d = tp[:, :, d:d + 1]
        poly = poly + jnp.abs(a_d - b_d)

    big_ref[0, :, :, :Q] = cls_cost + _PW * poly
    big_ref[0, :, :, Q:] = jnp.full((B, T, _QP - Q), jnp.inf, jnp.float32)
    big_ref[1, :, :, :Q] = lp
    big_ref[1, :, :, Q:] = jnp.zeros((B, T, _QP - Q), jnp.float32)
    bg_ref[:, :, :Q] = bg_lp
    bg_ref[:, :, Q:] = jnp.zeros((B, 1, _QP - Q), jnp.float32)
    bg_ref[0:1, 0:1, _QP - 1:_QP] = jnp.reshape(ce_bg, (1, 1, 1))


def _sc_match(big, bg):
    _, B, T, QP = big.shape
    nj = QP // 16
    mesh = plsc.VectorSubcoreMesh(core_axis_name="c", subcore_axis_name="s")

    @functools.partial(
        pl.kernel, mesh=mesh,
        compiler_params=pltpu.CompilerParams(needs_layout_passes=False),
        out_type=jax.ShapeDtypeStruct((B * 16,), jnp.float32),
        scratch_types=[
            pltpu.VMEM((T, QP), jnp.float32),
            pltpu.VMEM((T, QP), jnp.float32),
            pltpu.VMEM((1, QP), jnp.float32),
            pltpu.VMEM((16,), jnp.float32),
            pltpu.SemaphoreType.DMA,
            pltpu.SemaphoreType.DMA,
            pltpu.SemaphoreType.DMA,
        ],
    )
    def k(big_hbm, bg_hbm, out_hbm, c_v, lp_v, bg_v, out_v, s1, s2, s3):
        cid = lax.axis_index("c")
        sid = lax.axis_index("s")
        w = sid * 2 + cid

        @pl.when(w < B)
        def _():
            cp1 = pltpu.async_copy(big_hbm.at[0, w], c_v, s1)
            cp2 = pltpu.async_copy(big_hbm.at[1, w], lp_v, s2)
            cp3 = pltpu.async_copy(bg_hbm.at[w], bg_v, s3)
            cp1.wait()
            cp2.wait()
            cp3.wait()
            il = lax.iota(jnp.int32, 16)
            z16 = jnp.zeros((16,), jnp.float32)

            def gstep(t, carry):
                pacc, cacc, used = carry
                bestv = jnp.full((16,), jnp.inf, jnp.float32)
                besti = jnp.full((16,), jnp.int32(2 ** 30))
                for j in range(nj):
                    mv = c_v[t, pl.ds(16 * j, 16)] + used[j]
                    upd = mv < bestv
                    bestv = jnp.where(upd, mv, bestv)
                    besti = jnp.where(upd, 16 * j + il, besti)
                mn = jnp.min(bestv)
                cand = jnp.where(bestv == mn, besti, jnp.int32(2 ** 30))
                i = jnp.min(cand)
                jstar = i // 16
                selv = il == (i - 16 * jstar)
                used = tuple(
                    jnp.where(jstar == j,
                              jnp.where(selv, jnp.float32(jnp.inf), used[j]),
                              used[j])
                    for j in range(nj))
                lpc = lp_v[t, pl.ds(16 * jstar, 16)]
                bgc = bg_v[0, pl.ds(16 * jstar, 16)]
                pacc = pacc + jnp.where(
                    selv, (mn + jnp.exp(lpc)) * jnp.float32(1.0 / _PW), 0.0)
                cacc = cacc + jnp.where(selv, bgc - lpc, 0.0)
                return pacc, cacc, used

            used0 = tuple(z16 for _ in range(nj))
            pacc, cacc, _ = lax.fori_loop(0, T, gstep, (z16, z16, used0))
            last = bg_v[0, pl.ds(QP - 16, 16)]
            cacc = cacc + jnp.where(il == 15, last, 0.0)
            csum = jnp.sum(cacc)
            psum = jnp.sum(pacc)
            out_v[...] = jnp.where(il == 0, csum,
                                   jnp.where(il == 1, psum, 0.0))
            pltpu.sync_copy(out_v, out_hbm.at[pl.ds(w * 16, 16)])

    return k(big, bg)


def kernel(pred_logits, pred_polylines, tgt_labels, tgt_polylines):
    B, Q, C = pred_logits.shape
    T = tgt_labels.shape[1]
    big, bgm = pl.pallas_call(
        _dense_body,
        out_shape=[
            jax.ShapeDtypeStruct((2, B, T, _QP), jnp.float32),
            jax.ShapeDtypeStruct((B, 1, _QP), jnp.float32),
        ],
        scratch_shapes=[
            pltpu.VMEM((B, T + 2, Q), jnp.float32),
            pltpu.VMEM((B, pred_polylines.shape[2], Q), jnp.float32),
        ],
    )(pred_logits, pred_polylines, tgt_labels.astype(jnp.int32), tgt_polylines)
    parts = _sc_match(big, bgm).reshape(B, 16)
    loss_ce = jnp.sum(parts[:, 0]) / jnp.float32(B * Q)
    loss_poly = jnp.sum(parts[:, 1]) / jnp.float32(B * T)
    return jnp.stack([loss_ce, loss_poly])


# speedup vs baseline: 1.2938x; 1.0643x over previous
import functools

import jax
import jax.numpy as jnp
from jax import lax
from jax.experimental import pallas as pl
from jax.experimental.pallas import tpu as pltpu
from jax.experimental.pallas import tpu_sc as plsc

_NC = 50
_PW = 5.0
_QP = 304


def _dense_body(x_ref, pp_ref, lab_ref, tp_ref, big_ref, xl_scr, ppt_scr):
    B, Q, C = x_ref.shape
    T = lab_ref.shape[1]
    D = pp_ref.shape[2]

    x = x_ref[...]
    m = jnp.max(x, axis=2, keepdims=True)
    e = jnp.exp(x - m)

    labs = lab_ref[...]
    ci = lax.broadcasted_iota(jnp.int32, (B, T + 2, C), 2)
    ri = lax.broadcasted_iota(jnp.int32, (B, T + 2, C), 1)
    oh_lab = (ci == jnp.pad(labs, ((0, 0), (0, 2)))[:, :, None]).astype(jnp.float32)
    oh_bg = (ci == _NC).astype(jnp.float32)
    sel = jnp.where(ri < T, oh_lab, jnp.where(ri == T, 1.0, oh_bg))
    for b in range(B):
        xl_scr[b] = lax.dot_general(sel[b], e[b], (((1,), (1,)), ((), ())),
                                    preferred_element_type=jnp.float32)
    exl = xl_scr[:, :T, :]
    s_row = xl_scr[:, T:T + 1, :]
    ebg = xl_scr[:, T + 1:T + 2, :]

    cls_cost = -(exl / s_row)
    logls = jnp.log(s_row)
    lp = jnp.log(exl) - logls
    bg_lp = jnp.log(ebg) - logls
    ce_bg = -jnp.sum(bg_lp)

    di = lax.broadcasted_iota(jnp.int32, (D, D), 0)
    dj = lax.broadcasted_iota(jnp.int32, (D, D), 1)
    ident = (di == dj).astype(jnp.float32)
    pp = pp_ref[...]
    for b in range(B):
        ppt_scr[b] = lax.dot_general(ident, pp[b], (((1,), (1,)), ((), ())),
                                     preferred_element_type=jnp.float32)

    tp = tp_ref[...]
    poly = jnp.zeros((B, T, Q), dtype=jnp.float32)
    for d in range(D):
        a_d = ppt_scr[:, d:d + 1, :]
        b_d = tp[:, :, d:d + 1]
        poly = poly + jnp.abs(a_d - b_d)

    big_ref[0, :, :T, :Q] = cls_cost + _PW * poly
    big_ref[0, :, :T, Q:] = jnp.full((B, T, _QP - Q), jnp.inf, jnp.float32)
    big_ref[0, :, T:, :] = jnp.full((B, 1, _QP), jnp.inf, jnp.float32)
    big_ref[1, :, :T, :Q] = lp
    big_ref[1, :, :T, Q:] = jnp.zeros((B, T, _QP - Q), jnp.float32)
    big_ref[1, :, T:, :Q] = bg_lp
    big_ref[1, :, T:, Q:] = jnp.zeros((B, 1, _QP - Q), jnp.float32)
    big_ref[1:2, 0:1, T:, _QP - 1:_QP] = jnp.reshape(ce_bg, (1, 1, 1, 1))


def _sc_match(big, ce_scale, poly_scale):
    _, B, T1, QP = big.shape
    T = T1 - 1
    nj = QP // 16
    mesh = plsc.VectorSubcoreMesh(core_axis_name="c", subcore_axis_name="s")

    @functools.partial(
        pl.kernel, mesh=mesh,
        compiler_params=pltpu.CompilerParams(needs_layout_passes=False),
        out_type=[
            jax.ShapeDtypeStruct((B * 16,), jnp.float32),
            jax.ShapeDtypeStruct((2,), jnp.float32),
        ],
        scratch_types=[
            pltpu.VMEM((T1, QP), jnp.float32),
            pltpu.VMEM((T1, QP), jnp.float32),
            pltpu.VMEM((B * 16,), jnp.float32),
            pltpu.VMEM((16,), jnp.float32),
            pltpu.SemaphoreType.DMA,
            pltpu.SemaphoreType.DMA,
        ],
    )
    def k(big_hbm, part_hbm, fin_hbm, c_v, lp_v, stag_v, out_v, s1, s2):
        cid = lax.axis_index("c")
        sid = lax.axis_index("s")

        @pl.when(cid == 0)
        def _():
            w = sid
            cp1 = pltpu.async_copy(big_hbm.at[0, w], c_v, s1)
            cp2 = pltpu.async_copy(big_hbm.at[1, w], lp_v, s2)
            cp1.wait()
            cp2.wait()
            il = lax.iota(jnp.int32, 16)
            z16 = jnp.zeros((16,), jnp.float32)

            def gstep(t, carry):
                pacc, cacc, used = carry
                bestv = jnp.full((16,), jnp.inf, jnp.float32)
                besti = jnp.full((16,), jnp.int32(2 ** 30))
                for j in range(nj):
                    mv = c_v[t, pl.ds(16 * j, 16)] + used[j]
                    upd = mv < bestv
                    bestv = jnp.where(upd, mv, bestv)
                    besti = jnp.where(upd, 16 * j + il, besti)
                mn = jnp.min(bestv)
                cand = jnp.where(bestv == mn, besti, jnp.int32(2 ** 30))
                i = jnp.min(cand)
                jstar = i // 16
                selv = il == (i - 16 * jstar)
                used = tuple(
                    jnp.where(jstar == j,
                              jnp.where(selv, jnp.float32(jnp.inf), used[j]),
                              used[j])
                    for j in range(nj))
                lpc = lp_v[t, pl.ds(16 * jstar, 16)]
                bgc = lp_v[T, pl.ds(16 * jstar, 16)]
                pacc = pacc + jnp.where(
                    selv, (mn + jnp.exp(lpc)) * jnp.float32(1.0 / _PW), 0.0)
                cacc = cacc + jnp.where(selv, bgc - lpc, 0.0)
                return pacc, cacc, used

            used0 = tuple(z16 for _ in range(nj))
            pacc, cacc, _ = lax.fori_loop(0, T, gstep, (z16, z16, used0))
            last = lp_v[T, pl.ds(QP - 16, 16)]
            cacc = cacc + jnp.where(il == 15, last, 0.0)
            csum = jnp.sum(cacc)
            psum = jnp.sum(pacc)
            out_v[...] = jnp.where(il == 0, csum,
                                   jnp.where(il == 1, psum, 0.0))
            pltpu.sync_copy(out_v, part_hbm.at[pl.ds(w * 16, 16)])
            plsc.subcore_barrier()

            @pl.when(sid == 0)
            def _():
                pltpu.sync_copy(part_hbm, stag_v)
                acc = jnp.zeros((16,), jnp.float32)
                for g in range(B):
                    acc = acc + stag_v[pl.ds(g * 16, 16)]
                scale = jnp.where(il == 0, jnp.float32(ce_scale),
                                  jnp.where(il == 1, jnp.float32(poly_scale),
                                            0.0))
                out_v[...] = acc * scale
                pltpu.sync_copy(out_v.at[pl.ds(0, 2)], fin_hbm)

    return k(big)


def kernel(pred_logits, pred_polylines, tgt_labels, tgt_polylines):
    B, Q, C = pred_logits.shape
    T = tgt_labels.shape[1]
    big = pl.pallas_call(
        _dense_body,
        out_shape=jax.ShapeDtypeStruct((2, B, T + 1, _QP), jnp.float32),
        scratch_shapes=[
            pltpu.VMEM((B, T + 2, Q), jnp.float32),
            pltpu.VMEM((B, pred_polylines.shape[2], Q), jnp.float32),
        ],
    )(pred_logits, pred_polylines, tgt_labels.astype(jnp.int32), tgt_polylines)
    _, fin = _sc_match(big, 1.0 / (B * Q), 1.0 / (B * T))
    return fin


# speedup vs baseline: 1.3347x; 1.0316x over previous
import functools

import jax
import jax.numpy as jnp
from jax import lax
from jax.experimental import pallas as pl
from jax.experimental.pallas import tpu as pltpu
from jax.experimental.pallas import tpu_sc as plsc

_NC = 50
_PW = 5.0
_QP = 304


def _dense_body(x_ref, pp_ref, lab_ref, tp_ref, big_ref, xl_scr, ppt_scr):
    B, Q, C = x_ref.shape
    T = lab_ref.shape[1]
    D = pp_ref.shape[2]

    x = x_ref[...]
    m = jnp.max(x, axis=2, keepdims=True)
    e = jnp.exp(x - m)

    labs = lab_ref[...]
    ci = lax.broadcasted_iota(jnp.int32, (B, T + 2, C), 2)
    ri = lax.broadcasted_iota(jnp.int32, (B, T + 2, C), 1)
    oh_lab = (ci == jnp.pad(labs, ((0, 0), (0, 2)))[:, :, None]).astype(jnp.float32)
    oh_bg = (ci == _NC).astype(jnp.float32)
    sel = jnp.where(ri < T, oh_lab, jnp.where(ri == T, 1.0, oh_bg))
    for b in range(B):
        xl_scr[b] = lax.dot_general(sel[b], e[b], (((1,), (1,)), ((), ())),
                                    preferred_element_type=jnp.float32)
    exl = xl_scr[:, :T, :]
    s_row = xl_scr[:, T:T + 1, :]
    ebg = xl_scr[:, T + 1:T + 2, :]

    cls_cost = -(exl / s_row)
    logls = jnp.log(s_row)
    lp = jnp.log(exl) - logls
    bg_lp = jnp.log(ebg) - logls
    ce_bg = -jnp.sum(bg_lp)

    di = lax.broadcasted_iota(jnp.int32, (D, D), 0)
    dj = lax.broadcasted_iota(jnp.int32, (D, D), 1)
    ident = (di == dj).astype(jnp.float32)
    pp = pp_ref[...]
    for b in range(B):
        ppt_scr[b] = lax.dot_general(ident, pp[b], (((1,), (1,)), ((), ())),
                                     preferred_element_type=jnp.float32)

    tp = tp_ref[...]
    poly = jnp.zeros((B, T, Q), dtype=jnp.float32)
    for d in range(D):
        a_d = ppt_scr[:, d:d + 1, :]
        b_d = tp[:, :, d:d + 1]
        poly = poly + jnp.abs(a_d - b_d)

    big_ref[0, :, :T, :Q] = cls_cost + _PW * poly
    big_ref[0, :, :T, Q:] = jnp.full((B, T, _QP - Q), jnp.inf, jnp.float32)
    big_ref[0, :, T:, :] = jnp.full((B, 1, _QP), jnp.inf, jnp.float32)
    big_ref[1, :, :T, :Q] = lp
    big_ref[1, :, :T, Q:] = jnp.zeros((B, T, _QP - Q), jnp.float32)
    big_ref[1, :, T:, :Q] = bg_lp
    big_ref[1, :, T:, Q:] = jnp.zeros((B, 1, _QP - Q), jnp.float32)
    big_ref[1:2, 0:1, T:, _QP - 1:_QP] = jnp.reshape(ce_bg, (1, 1, 1, 1))


def _sc_match(big, ce_scale, poly_scale):
    _, B, T1, QP = big.shape
    T = T1 - 1
    nj = QP // 16
    mesh = plsc.VectorSubcoreMesh(core_axis_name="c", subcore_axis_name="s",
                                  num_cores=1)

    @functools.partial(
        pl.kernel, mesh=mesh,
        compiler_params=pltpu.CompilerParams(needs_layout_passes=False),
        out_type=[
            jax.ShapeDtypeStruct((B * 16,), jnp.float32),
            jax.ShapeDtypeStruct((2,), jnp.float32),
        ],
        scratch_types=[
            pltpu.VMEM((T1, QP), jnp.float32),
            pltpu.VMEM((T1, QP), jnp.float32),
            pltpu.VMEM((B * 16,), jnp.float32),
            pltpu.VMEM((16,), jnp.float32),
            pltpu.SemaphoreType.DMA,
            pltpu.SemaphoreType.DMA,
        ],
    )
    def k(big_hbm, part_hbm, fin_hbm, c_v, lp_v, stag_v, out_v, s1, s2):
        cid = lax.axis_index("c")
        sid = lax.axis_index("s")

        @pl.when(cid == 0)
        def _():
            w = sid
            cp1 = pltpu.async_copy(big_hbm.at[0, w], c_v, s1)
            cp2 = pltpu.async_copy(big_hbm.at[1, w], lp_v, s2)
            cp1.wait()
            cp2.wait()
            il = lax.iota(jnp.int32, 16)
            z16 = jnp.zeros((16,), jnp.float32)

            def gstep(t, carry):
                pacc, cacc, used = carry
                bestv = jnp.full((16,), jnp.inf, jnp.float32)
                besti = jnp.full((16,), jnp.int32(2 ** 30))
                for j in range(nj):
                    mv = c_v[t, pl.ds(16 * j, 16)] + used[j]
                    upd = mv < bestv
                    bestv = jnp.where(upd, mv, bestv)
                    besti = jnp.where(upd, 16 * j + il, besti)
                mn = jnp.min(bestv)
                cand = jnp.where(bestv == mn, besti, jnp.int32(2 ** 30))
                i = jnp.min(cand)
                jstar = i // 16
                selv = il == (i - 16 * jstar)
                used = tuple(
                    jnp.where(jstar == j,
                              jnp.where(selv, jnp.float32(jnp.inf), used[j]),
                              used[j])
                    for j in range(nj))
                lpc = lp_v[t, pl.ds(16 * jstar, 16)]
                bgc = lp_v[T, pl.ds(16 * jstar, 16)]
                pacc = pacc + jnp.where(
                    selv, (mn + jnp.exp(lpc)) * jnp.float32(1.0 / _PW), 0.0)
                cacc = cacc + jnp.where(selv, bgc - lpc, 0.0)
                return pacc, cacc, used

            used0 = tuple(z16 for _ in range(nj))
            pacc, cacc, _ = lax.fori_loop(0, T, gstep, (z16, z16, used0))
            last = lp_v[T, pl.ds(QP - 16, 16)]
            cacc = cacc + jnp.where(il == 15, last, 0.0)
            csum = jnp.sum(cacc)
            psum = jnp.sum(pacc)
            out_v[...] = jnp.where(il == 0, csum,
                                   jnp.where(il == 1, psum, 0.0))
            pltpu.sync_copy(out_v, part_hbm.at[pl.ds(w * 16, 16)])
            plsc.subcore_barrier()

            @pl.when(sid == 0)
            def _():
                pltpu.sync_copy(part_hbm, stag_v)
                acc = jnp.zeros((16,), jnp.float32)
                for g in range(B):
                    acc = acc + stag_v[pl.ds(g * 16, 16)]
                scale = jnp.where(il == 0, jnp.float32(ce_scale),
                                  jnp.where(il == 1, jnp.float32(poly_scale),
                                            0.0))
                out_v[...] = acc * scale
                pltpu.sync_copy(out_v.at[pl.ds(0, 2)], fin_hbm)

    return k(big)


def kernel(pred_logits, pred_polylines, tgt_labels, tgt_polylines):
    B, Q, C = pred_logits.shape
    T = tgt_labels.shape[1]
    big = pl.pallas_call(
        _dense_body,
        out_shape=jax.ShapeDtypeStruct((2, B, T + 1, _QP), jnp.float32),
        scratch_shapes=[
            pltpu.VMEM((B, T + 2, Q), jnp.float32),
            pltpu.VMEM((B, pred_polylines.shape[2], Q), jnp.float32),
        ],
    )(pred_logits, pred_polylines, tgt_labels.astype(jnp.int32), tgt_polylines)
    _, fin = _sc_match(big, 1.0 / (B * Q), 1.0 / (B * T))
    return fin
